# Initial kernel scaffold; baseline (speedup 1.0000x reference)
import dataclasses
import functools

import jax
import jax.numpy as jnp
from jax import lax
from jax.experimental import pallas as pl
from jax.experimental.pallas import tpu as pltpu
from jax.experimental.pallas import tpu_sc as plsc

N_NODES = 100000
N_DIH = 3200000
N_TYPES = 26
N_GRAPHS = 64
N_DEGS = 3
R_TAB = N_TYPES ** 4

NC, NS, L = 2, 16, 16
NW = NC * NS
W_PER_TILE = N_DIH // NW
B = 800
N_CHUNKS = W_PER_TILE // B
NV = B // L

PREP_CB = 8192
R_PAD = 458752


def _prep_body(th_ref, k_ref, kc_ref, ks_ref, ksum_ref):
    th = th_ref[...]
    k = k_ref[...]
    kc_ref[...] = k * jnp.cos(th)
    ks_ref[...] = k * jnp.sin(th)
    ksum_ref[...] = jnp.sum(k, axis=0, keepdims=True)


_prep = pl.pallas_call(
    _prep_body,
    grid=(R_PAD // PREP_CB,),
    in_specs=[
        pl.BlockSpec((N_DEGS, PREP_CB), lambda i: (0, i)),
        pl.BlockSpec((N_DEGS, PREP_CB), lambda i: (0, i)),
    ],
    out_specs=[
        pl.BlockSpec((N_DEGS, PREP_CB), lambda i: (0, i)),
        pl.BlockSpec((N_DEGS, PREP_CB), lambda i: (0, i)),
        pl.BlockSpec((1, PREP_CB), lambda i: (0, i)),
    ],
    out_shape=[
        jax.ShapeDtypeStruct((N_DEGS, R_PAD), jnp.float32),
        jax.ShapeDtypeStruct((N_DEGS, R_PAD), jnp.float32),
        jax.ShapeDtypeStruct((1, R_PAD), jnp.float32),
    ],
)


def _final_body(p_ref, o_ref):
    o_ref[...] = jnp.sum(p_ref[...], axis=0, keepdims=True)


_final = pl.pallas_call(
    _final_body,
    out_shape=jax.ShapeDtypeStruct((1, N_GRAPHS), jnp.float32),
)


def _rsqrt(x):
    i = plsc.bitcast(x, jnp.int32)
    i = jnp.int32(0x5F3759DF) - (i >> 1)
    y = plsc.bitcast(i, jnp.float32)
    hx = x * jnp.float32(0.5)
    for _ in range(3):
        y = y * (jnp.float32(1.5) - hx * y * y)
    return y


def _splat_i32(val):
    return jnp.full((L,), val, jnp.int32)


def _sc_body(m0, m1, m2, m3, mb, nodes, ptab, out,
             i0, i1, i2, i3, gbuf, fidx, nb0, nb1, nb2, nb3, pbuf, acc,
             sem):
    cid = lax.axis_index("c")
    sid = lax.axis_index("s")
    wid = sid * NC + cid
    base0 = wid * W_PER_TILE

    for r in range(L):
        for cb in range(N_GRAPHS // L):
            acc[r, pl.ds(cb * L, L)] = jnp.zeros((L,), jnp.float32)

    lane = lax.iota(jnp.int32, L)
    col0 = _splat_i32(0)
    col1 = _splat_i32(1)
    col2 = _splat_i32(2)
    col3 = _splat_i32(3)
    col4 = _splat_i32(4)
    col5 = _splat_i32(5)
    col6 = _splat_i32(6)

    @pl.loop(0, N_CHUNKS)
    def _chunk(ci):
        base = base0 + ci * B
        pltpu.sync_copy(m0.at[pl.ds(base, B)], i0)
        pltpu.sync_copy(m1.at[pl.ds(base, B)], i1)
        pltpu.sync_copy(m2.at[pl.ds(base, B)], i2)
        pltpu.sync_copy(m3.at[pl.ds(base, B)], i3)
        pltpu.sync_copy(mb.at[pl.ds(base, B)], gbuf)

        g0 = pltpu.async_copy(nodes.at[i0], nb0, sem)
        g1 = pltpu.async_copy(nodes.at[i1], nb1, sem)
        g2 = pltpu.async_copy(nodes.at[i2], nb2, sem)
        g3 = pltpu.async_copy(nodes.at[i3], nb3, sem)
        g0.wait()
        g1.wait()
        g2.wait()
        g3.wait()

        @pl.loop(0, NV)
        def _p1(v):
            rows = lane + v * L
            t0 = plsc.load_gather(nb0, [rows, col3])
            t1 = plsc.load_gather(nb1, [rows, col3])
            t2 = plsc.load_gather(nb2, [rows, col3])
            t3 = plsc.load_gather(nb3, [rows, col3])
            f = ((t0 * jnp.float32(N_TYPES) + t1) * jnp.float32(N_TYPES)
                 + t2) * jnp.float32(N_TYPES) + t3
            fidx[pl.ds(v * L, L)] = f.astype(jnp.int32)

        pltpu.async_copy(ptab.at[fidx], pbuf, sem).wait()

        @pl.loop(0, NV)
        def _p2(v):
            rows = lane + v * L
            p0x = plsc.load_gather(nb0, [rows, col0])
            p0y = plsc.load_gather(nb0, [rows, col1])
            p0z = plsc.load_gather(nb0, [rows, col2])
            p1x = plsc.load_gather(nb1, [rows, col0])
            p1y = plsc.load_gather(nb1, [rows, col1])
            p1z = plsc.load_gather(nb1, [rows, col2])
            p2x = plsc.load_gather(nb2, [rows, col0])
            p2y = plsc.load_gather(nb2, [rows, col1])
            p2z = plsc.load_gather(nb2, [rows, col2])
            p3x = plsc.load_gather(nb3, [rows, col0])
            p3y = plsc.load_gather(nb3, [rows, col1])
            p3z = plsc.load_gather(nb3, [rows, col2])

            b1x, b1y, b1z = p1x - p0x, p1y - p0y, p1z - p0z
            b2x, b2y, b2z = p2x - p1x, p2y - p1y, p2z - p1z
            b3x, b3y, b3z = p3x - p2x, p3y - p2y, p3z - p2z

            n1x = b1y * b2z - b1z * b2y
            n1y = b1z * b2x - b1x * b2z
            n1z = b1x * b2y - b1y * b2x
            n2x = b2y * b3z - b2z * b3y
            n2y = b2z * b3x - b2x * b3z
            n2z = b2x * b3y - b2y * b3x
            mx = n1y * b2z - n1z * b2y
            my = n1z * b2x - n1x * b2z
            mz = n1x * b2y - n1y * b2x

            x = n1x * n2x + n1y * n2y + n1z * n2z
            yp = mx * n2x + my * n2y + mz * n2z
            s2 = jnp.maximum(b2x * b2x + b2y * b2y + b2z * b2z,
                             jnp.float32(1e-30))
            y = yp * _rsqrt(s2)
            r2 = x * x + y * y
            w = _rsqrt(jnp.maximum(r2, jnp.float32(1e-30)))
            deg = r2 < jnp.float32(1e-30)
            c = jnp.where(deg, jnp.float32(1.0), x * w)
            s = jnp.where(deg, jnp.float32(0.0), y * w)

            c2 = jnp.float32(2.0) * c * c - jnp.float32(1.0)
            s2t = jnp.float32(2.0) * s * c
            c3 = jnp.float32(2.0) * c * c2 - c
            s3 = jnp.float32(2.0) * c * s2t - s

            kc1 = plsc.load_gather(pbuf, [rows, col0])
            kc2 = plsc.load_gather(pbuf, [rows, col1])
            kc3 = plsc.load_gather(pbuf, [rows, col2])
            ks1 = plsc.load_gather(pbuf, [rows, col3])
            ks2 = plsc.load_gather(pbuf, [rows, col4])
            ks3 = plsc.load_gather(pbuf, [rows, col5])
            ksm = plsc.load_gather(pbuf, [rows, col6])

            V = ksm - (kc1 * c + kc2 * c2 + kc3 * c3
                       + ks1 * s + ks2 * s2t + ks3 * s3)

            g = gbuf[pl.ds(v * L, L)]
            plsc.addupdate_scatter(acc, [lane, g], V)

    pltpu.sync_copy(acc, out.at[wid])


def _make_sc_kernel():
    mesh = plsc.VectorSubcoreMesh(core_axis_name="c", subcore_axis_name="s")
    cp = pltpu.CompilerParams()
    if "needs_layout_passes" in pltpu.CompilerParams.__dataclass_fields__:
        cp = dataclasses.replace(cp, needs_layout_passes=False)
    return pl.kernel(
        _sc_body,
        mesh=mesh,
        out_type=jax.ShapeDtypeStruct((NW, L, N_GRAPHS), jnp.float32),
        scratch_types=[
            pltpu.VMEM((B,), jnp.int32),
            pltpu.VMEM((B,), jnp.int32),
            pltpu.VMEM((B,), jnp.int32),
            pltpu.VMEM((B,), jnp.int32),
            pltpu.VMEM((B,), jnp.int32),
            pltpu.VMEM((B,), jnp.int32),
            pltpu.VMEM((B, 4), jnp.float32),
            pltpu.VMEM((B, 4), jnp.float32),
            pltpu.VMEM((B, 4), jnp.float32),
            pltpu.VMEM((B, 4), jnp.float32),
            pltpu.VMEM((B, 8), jnp.float32),
            pltpu.VMEM((L, N_GRAPHS), jnp.float32),
            pltpu.SemaphoreType.DMA,
        ],
        compiler_params=cp,
    )


_sc_kernel = _make_sc_kernel()


def kernel(pos, mapping, mapping_batch, atom_types, thetas, ks):
    f32 = jnp.float32
    th = jnp.pad(thetas.reshape(R_TAB, N_DEGS).T.astype(f32),
                 ((0, 0), (0, R_PAD - R_TAB)))
    kk = jnp.pad(ks.reshape(R_TAB, N_DEGS).T.astype(f32),
                 ((0, 0), (0, R_PAD - R_TAB)))
    kc, ksn, ksum = _prep(th, kk)
    ptab = jnp.concatenate(
        [kc[:, :R_TAB].T, ksn[:, :R_TAB].T, ksum[:1, :R_TAB].T,
         jnp.zeros((R_TAB, 1), f32)], axis=1)

    nodes = jnp.concatenate(
        [pos.astype(f32), atom_types.astype(f32)[:, None]], axis=1)

    mapping = mapping.astype(jnp.int32)
    m0, m1, m2, m3 = mapping[0], mapping[1], mapping[2], mapping[3]
    mb = mapping_batch.astype(jnp.int32)

    partials = _sc_kernel(m0, m1, m2, m3, mb, nodes, ptab)
    y = _final(partials.reshape(NW * L, N_GRAPHS))
    return y[0]


# speedup vs baseline: 127.2853x; 127.2853x over previous
import dataclasses
import functools

import jax
import jax.numpy as jnp
from jax import lax
from jax.experimental import pallas as pl
from jax.experimental.pallas import tpu as pltpu
from jax.experimental.pallas import tpu_sc as plsc

N_NODES = 100000
N_DIH = 3200000
N_TYPES = 26
N_GRAPHS = 64
N_DEGS = 3
R_TAB = N_TYPES ** 4

NC, NS, L = 2, 16, 16
NW = NC * NS
W_PER_TILE = N_DIH // NW
B = 800
N_CHUNKS = W_PER_TILE // B
NV = B // L

PREP_CB = 8192
R_PAD = 458752


def _prep_body(th_ref, k_ref, kc_ref, ks_ref, ksum_ref):
    th = th_ref[...]
    k = k_ref[...]
    kc_ref[...] = k * jnp.cos(th)
    ks_ref[...] = k * jnp.sin(th)
    ksum_ref[...] = jnp.sum(k, axis=0, keepdims=True)


_prep = pl.pallas_call(
    _prep_body,
    grid=(R_PAD // PREP_CB,),
    in_specs=[
        pl.BlockSpec((N_DEGS, PREP_CB), lambda i: (0, i)),
        pl.BlockSpec((N_DEGS, PREP_CB), lambda i: (0, i)),
    ],
    out_specs=[
        pl.BlockSpec((N_DEGS, PREP_CB), lambda i: (0, i)),
        pl.BlockSpec((N_DEGS, PREP_CB), lambda i: (0, i)),
        pl.BlockSpec((1, PREP_CB), lambda i: (0, i)),
    ],
    out_shape=[
        jax.ShapeDtypeStruct((N_DEGS, R_PAD), jnp.float32),
        jax.ShapeDtypeStruct((N_DEGS, R_PAD), jnp.float32),
        jax.ShapeDtypeStruct((1, R_PAD), jnp.float32),
    ],
)


def _final_body(p_ref, o_ref):
    o_ref[...] = jnp.sum(p_ref[...], axis=0, keepdims=True)


_final = pl.pallas_call(
    _final_body,
    out_shape=jax.ShapeDtypeStruct((1, N_GRAPHS), jnp.float32),
)


def _rsqrt(x):
    i = plsc.bitcast(x, jnp.int32)
    i = jnp.int32(0x5F3759DF) - (i >> 1)
    y = plsc.bitcast(i, jnp.float32)
    hx = x * jnp.float32(0.5)
    for _ in range(3):
        y = y * (jnp.float32(1.5) - hx * y * y)
    return y


def _splat_i32(val):
    return jnp.full((L,), val, jnp.int32)


def _sc_body(m0, m1, m2, m3, mb, nodes, ptab, out,
             i0, i1, i2, i3, gbuf, fidx, nb0, nb1, nb2, nb3, pbuf, acc,
             sem):
    cid = lax.axis_index("c")
    sid = lax.axis_index("s")
    wid = sid * NC + cid
    base0 = wid * W_PER_TILE

    for r in range(L):
        for cb in range(N_GRAPHS // L):
            acc[r, pl.ds(cb * L, L)] = jnp.zeros((L,), jnp.float32)

    lane = lax.iota(jnp.int32, L)
    col0 = _splat_i32(0)
    col1 = _splat_i32(1)
    col2 = _splat_i32(2)
    col3 = _splat_i32(3)
    col4 = _splat_i32(4)
    col5 = _splat_i32(5)
    col6 = _splat_i32(6)

    @pl.loop(0, N_CHUNKS)
    def _chunk(ci):
        base = base0 + ci * B
        pltpu.sync_copy(m0.at[pl.ds(base, B)], i0)
        pltpu.sync_copy(m1.at[pl.ds(base, B)], i1)
        pltpu.sync_copy(m2.at[pl.ds(base, B)], i2)
        pltpu.sync_copy(m3.at[pl.ds(base, B)], i3)
        pltpu.sync_copy(mb.at[pl.ds(base, B)], gbuf)

        pltpu.async_copy(nodes.at[i0], nb0, sem).wait()
        pltpu.async_copy(nodes.at[i1], nb1, sem).wait()
        pltpu.async_copy(nodes.at[i2], nb2, sem).wait()
        pltpu.async_copy(nodes.at[i3], nb3, sem).wait()

        @pl.loop(0, NV)
        def _p1(v):
            rows = lane + v * L
            t0 = plsc.load_gather(nb0, [rows, col3])
            t1 = plsc.load_gather(nb1, [rows, col3])
            t2 = plsc.load_gather(nb2, [rows, col3])
            t3 = plsc.load_gather(nb3, [rows, col3])
            f = ((t0 * jnp.float32(N_TYPES) + t1) * jnp.float32(N_TYPES)
                 + t2) * jnp.float32(N_TYPES) + t3
            f = jnp.minimum(jnp.maximum(f, jnp.float32(0.0)),
                            jnp.float32(R_TAB - 1))
            fidx[pl.ds(v * L, L)] = f.astype(jnp.int32)

        pltpu.async_copy(ptab.at[fidx], pbuf, sem).wait()

        @pl.loop(0, NV)
        def _p2(v):
            rows = lane + v * L
            p0x = plsc.load_gather(nb0, [rows, col0])
            p0y = plsc.load_gather(nb0, [rows, col1])
            p0z = plsc.load_gather(nb0, [rows, col2])
            p1x = plsc.load_gather(nb1, [rows, col0])
            p1y = plsc.load_gather(nb1, [rows, col1])
            p1z = plsc.load_gather(nb1, [rows, col2])
            p2x = plsc.load_gather(nb2, [rows, col0])
            p2y = plsc.load_gather(nb2, [rows, col1])
            p2z = plsc.load_gather(nb2, [rows, col2])
            p3x = plsc.load_gather(nb3, [rows, col0])
            p3y = plsc.load_gather(nb3, [rows, col1])
            p3z = plsc.load_gather(nb3, [rows, col2])

            b1x, b1y, b1z = p1x - p0x, p1y - p0y, p1z - p0z
            b2x, b2y, b2z = p2x - p1x, p2y - p1y, p2z - p1z
            b3x, b3y, b3z = p3x - p2x, p3y - p2y, p3z - p2z

            n1x = b1y * b2z - b1z * b2y
            n1y = b1z * b2x - b1x * b2z
            n1z = b1x * b2y - b1y * b2x
            n2x = b2y * b3z - b2z * b3y
            n2y = b2z * b3x - b2x * b3z
            n2z = b2x * b3y - b2y * b3x
            mx = n1y * b2z - n1z * b2y
            my = n1z * b2x - n1x * b2z
            mz = n1x * b2y - n1y * b2x

            x = n1x * n2x + n1y * n2y + n1z * n2z
            yp = mx * n2x + my * n2y + mz * n2z
            s2 = jnp.maximum(b2x * b2x + b2y * b2y + b2z * b2z,
                             jnp.float32(1e-30))
            y = yp * _rsqrt(s2)
            r2 = x * x + y * y
            w = _rsqrt(jnp.maximum(r2, jnp.float32(1e-30)))
            deg = r2 < jnp.float32(1e-30)
            c = jnp.where(deg, jnp.float32(1.0), x * w)
            s = jnp.where(deg, jnp.float32(0.0), y * w)

            c2 = jnp.float32(2.0) * c * c - jnp.float32(1.0)
            s2t = jnp.float32(2.0) * s * c
            c3 = jnp.float32(2.0) * c * c2 - c
            s3 = jnp.float32(2.0) * c * s2t - s

            kc1 = plsc.load_gather(pbuf, [rows, col0])
            kc2 = plsc.load_gather(pbuf, [rows, col1])
            kc3 = plsc.load_gather(pbuf, [rows, col2])
            ks1 = plsc.load_gather(pbuf, [rows, col3])
            ks2 = plsc.load_gather(pbuf, [rows, col4])
            ks3 = plsc.load_gather(pbuf, [rows, col5])
            ksm = plsc.load_gather(pbuf, [rows, col6])

            V = ksm - (kc1 * c + kc2 * c2 + kc3 * c3
                       + ks1 * s + ks2 * s2t + ks3 * s3)

            g = gbuf[pl.ds(v * L, L)]
            plsc.addupdate_scatter(acc, [lane, g], V)

    pltpu.sync_copy(acc, out.at[wid])


@functools.cache
def _make_sc_kernel():
    mesh = plsc.VectorSubcoreMesh(core_axis_name="c", subcore_axis_name="s")
    cp = pltpu.CompilerParams(use_tc_tiling_on_sc=False,
                              needs_layout_passes=False)
    return pl.kernel(
        _sc_body,
        mesh=mesh,
        out_type=jax.ShapeDtypeStruct((NW, L, N_GRAPHS), jnp.float32),
        scratch_types=[
            pltpu.VMEM((B,), jnp.int32),
            pltpu.VMEM((B,), jnp.int32),
            pltpu.VMEM((B,), jnp.int32),
            pltpu.VMEM((B,), jnp.int32),
            pltpu.VMEM((B,), jnp.int32),
            pltpu.VMEM((B,), jnp.int32),
            pltpu.VMEM((B, 8), jnp.float32),
            pltpu.VMEM((B, 8), jnp.float32),
            pltpu.VMEM((B, 8), jnp.float32),
            pltpu.VMEM((B, 8), jnp.float32),
            pltpu.VMEM((B, 8), jnp.float32),
            pltpu.VMEM((L, N_GRAPHS), jnp.float32),
            pltpu.SemaphoreType.DMA,
        ],
        compiler_params=cp,
    )


def kernel(pos, mapping, mapping_batch, atom_types, thetas, ks):
    f32 = jnp.float32
    th = jnp.pad(thetas.reshape(R_TAB, N_DEGS).T.astype(f32),
                 ((0, 0), (0, R_PAD - R_TAB)))
    kk = jnp.pad(ks.reshape(R_TAB, N_DEGS).T.astype(f32),
                 ((0, 0), (0, R_PAD - R_TAB)))
    kc, ksn, ksum = _prep(th, kk)
    ptab = jnp.concatenate(
        [kc[:, :R_TAB].T, ksn[:, :R_TAB].T, ksum[:1, :R_TAB].T,
         jnp.zeros((R_TAB, 1), f32)], axis=1)

    nodes = jnp.concatenate(
        [pos.astype(f32), atom_types.astype(f32)[:, None],
         jnp.zeros((N_NODES, 4), f32)], axis=1)

    mapping = mapping.astype(jnp.int32)
    m0, m1, m2, m3 = mapping[0], mapping[1], mapping[2], mapping[3]
    mb = mapping_batch.astype(jnp.int32)

    partials = _make_sc_kernel()(m0, m1, m2, m3, mb, nodes, ptab)
    y = _final(partials.reshape(NW * L, N_GRAPHS))
    return y[0]


# speedup vs baseline: 157.9463x; 1.2409x over previous
import dataclasses
import functools

import jax
import jax.numpy as jnp
from jax import lax
from jax.experimental import pallas as pl
from jax.experimental.pallas import tpu as pltpu
from jax.experimental.pallas import tpu_sc as plsc

N_NODES = 100000
N_DIH = 3200000
N_TYPES = 26
N_GRAPHS = 64
N_DEGS = 3
R_TAB = N_TYPES ** 4

NC, NS, L = 2, 16, 16
NW = NC * NS
W_PER_TILE = N_DIH // NW
B = 800
N_CHUNKS = W_PER_TILE // B
NV = B // L

PREP_CB = 8192
R_PAD = 458752


def _prep_body(th_ref, k_ref, kc_ref, ks_ref, ksum_ref):
    th = th_ref[...]
    k = k_ref[...]
    kc_ref[...] = k * jnp.cos(th)
    ks_ref[...] = k * jnp.sin(th)
    ksum_ref[...] = jnp.sum(k, axis=0, keepdims=True)


_prep = pl.pallas_call(
    _prep_body,
    grid=(R_PAD // PREP_CB,),
    in_specs=[
        pl.BlockSpec((N_DEGS, PREP_CB), lambda i: (0, i)),
        pl.BlockSpec((N_DEGS, PREP_CB), lambda i: (0, i)),
    ],
    out_specs=[
        pl.BlockSpec((N_DEGS, PREP_CB), lambda i: (0, i)),
        pl.BlockSpec((N_DEGS, PREP_CB), lambda i: (0, i)),
        pl.BlockSpec((1, PREP_CB), lambda i: (0, i)),
    ],
    out_shape=[
        jax.ShapeDtypeStruct((N_DEGS, R_PAD), jnp.float32),
        jax.ShapeDtypeStruct((N_DEGS, R_PAD), jnp.float32),
        jax.ShapeDtypeStruct((1, R_PAD), jnp.float32),
    ],
)


def _final_body(p_ref, o_ref):
    o_ref[...] = jnp.sum(p_ref[...], axis=0, keepdims=True)


_final = pl.pallas_call(
    _final_body,
    out_shape=jax.ShapeDtypeStruct((1, N_GRAPHS), jnp.float32),
)


def _rsqrt(x):
    i = plsc.bitcast(x, jnp.int32)
    i = jnp.int32(0x5F3759DF) - (i >> 1)
    y = plsc.bitcast(i, jnp.float32)
    hx = x * jnp.float32(0.5)
    for _ in range(3):
        y = y * (jnp.float32(1.5) - hx * y * y)
    return y


def _splat_i32(val):
    return jnp.full((L,), val, jnp.int32)


def _sc_body(m0, m1, m2, m3, mb, nodes, ptab, out,
             i0, i1, i2, i3, gbuf, fidx, nb0, nb1, nb2, nb3, pbuf, acc,
             sem):
    cid = lax.axis_index("c")
    sid = lax.axis_index("s")
    wid = sid * NC + cid
    base0 = wid * W_PER_TILE

    for r in range(L):
        for cb in range(N_GRAPHS // L):
            acc[r, pl.ds(cb * L, L)] = jnp.zeros((L,), jnp.float32)

    lane = lax.iota(jnp.int32, L)
    col0 = _splat_i32(0)
    col1 = _splat_i32(1)
    col2 = _splat_i32(2)
    col3 = _splat_i32(3)
    col4 = _splat_i32(4)
    col5 = _splat_i32(5)
    col6 = _splat_i32(6)

    @pl.loop(0, N_CHUNKS)
    def _chunk(ci):
        base = base0 + ci * B
        l0 = pltpu.async_copy(m0.at[pl.ds(base, B)], i0, sem)
        l1 = pltpu.async_copy(m1.at[pl.ds(base, B)], i1, sem)
        l2 = pltpu.async_copy(m2.at[pl.ds(base, B)], i2, sem)
        l3 = pltpu.async_copy(m3.at[pl.ds(base, B)], i3, sem)
        l4 = pltpu.async_copy(mb.at[pl.ds(base, B)], gbuf, sem)
        l0.wait()
        l1.wait()
        l2.wait()
        l3.wait()
        l4.wait()

        g0 = pltpu.async_copy(nodes.at[i0], nb0, sem)
        g1 = pltpu.async_copy(nodes.at[i1], nb1, sem)
        g2 = pltpu.async_copy(nodes.at[i2], nb2, sem)
        g3 = pltpu.async_copy(nodes.at[i3], nb3, sem)
        g0.wait()
        g1.wait()
        g2.wait()
        g3.wait()

        @pl.loop(0, NV)
        def _p1(v):
            rows = lane + v * L
            t0 = plsc.load_gather(nb0, [rows, col3])
            t1 = plsc.load_gather(nb1, [rows, col3])
            t2 = plsc.load_gather(nb2, [rows, col3])
            t3 = plsc.load_gather(nb3, [rows, col3])
            f = ((t0 * jnp.float32(N_TYPES) + t1) * jnp.float32(N_TYPES)
                 + t2) * jnp.float32(N_TYPES) + t3
            f = jnp.minimum(jnp.maximum(f, jnp.float32(0.0)),
                            jnp.float32(R_TAB - 1))
            fidx[pl.ds(v * L, L)] = f.astype(jnp.int32)

        pltpu.async_copy(ptab.at[fidx], pbuf, sem).wait()

        @pl.loop(0, NV)
        def _p2(v):
            rows = lane + v * L
            p0x = plsc.load_gather(nb0, [rows, col0])
            p0y = plsc.load_gather(nb0, [rows, col1])
            p0z = plsc.load_gather(nb0, [rows, col2])
            p1x = plsc.load_gather(nb1, [rows, col0])
            p1y = plsc.load_gather(nb1, [rows, col1])
            p1z = plsc.load_gather(nb1, [rows, col2])
            p2x = plsc.load_gather(nb2, [rows, col0])
            p2y = plsc.load_gather(nb2, [rows, col1])
            p2z = plsc.load_gather(nb2, [rows, col2])
            p3x = plsc.load_gather(nb3, [rows, col0])
            p3y = plsc.load_gather(nb3, [rows, col1])
            p3z = plsc.load_gather(nb3, [rows, col2])

            b1x, b1y, b1z = p1x - p0x, p1y - p0y, p1z - p0z
            b2x, b2y, b2z = p2x - p1x, p2y - p1y, p2z - p1z
            b3x, b3y, b3z = p3x - p2x, p3y - p2y, p3z - p2z

            n1x = b1y * b2z - b1z * b2y
            n1y = b1z * b2x - b1x * b2z
            n1z = b1x * b2y - b1y * b2x
            n2x = b2y * b3z - b2z * b3y
            n2y = b2z * b3x - b2x * b3z
            n2z = b2x * b3y - b2y * b3x
            mx = n1y * b2z - n1z * b2y
            my = n1z * b2x - n1x * b2z
            mz = n1x * b2y - n1y * b2x

            x = n1x * n2x + n1y * n2y + n1z * n2z
            yp = mx * n2x + my * n2y + mz * n2z
            s2 = jnp.maximum(b2x * b2x + b2y * b2y + b2z * b2z,
                             jnp.float32(1e-30))
            y = yp * _rsqrt(s2)
            r2 = x * x + y * y
            w = _rsqrt(jnp.maximum(r2, jnp.float32(1e-30)))
            deg = r2 < jnp.float32(1e-30)
            c = jnp.where(deg, jnp.float32(1.0), x * w)
            s = jnp.where(deg, jnp.float32(0.0), y * w)

            c2 = jnp.float32(2.0) * c * c - jnp.float32(1.0)
            s2t = jnp.float32(2.0) * s * c
            c3 = jnp.float32(2.0) * c * c2 - c
            s3 = jnp.float32(2.0) * c * s2t - s

            kc1 = plsc.load_gather(pbuf, [rows, col0])
            kc2 = plsc.load_gather(pbuf, [rows, col1])
            kc3 = plsc.load_gather(pbuf, [rows, col2])
            ks1 = plsc.load_gather(pbuf, [rows, col3])
            ks2 = plsc.load_gather(pbuf, [rows, col4])
            ks3 = plsc.load_gather(pbuf, [rows, col5])
            ksm = plsc.load_gather(pbuf, [rows, col6])

            V = ksm - (kc1 * c + kc2 * c2 + kc3 * c3
                       + ks1 * s + ks2 * s2t + ks3 * s3)

            g = gbuf[pl.ds(v * L, L)]
            plsc.addupdate_scatter(acc, [lane, g], V)

    pltpu.sync_copy(acc, out.at[wid])


@functools.cache
def _make_sc_kernel():
    mesh = plsc.VectorSubcoreMesh(core_axis_name="c", subcore_axis_name="s")
    cp = pltpu.CompilerParams(use_tc_tiling_on_sc=False,
                              needs_layout_passes=False)
    return pl.kernel(
        _sc_body,
        mesh=mesh,
        out_type=jax.ShapeDtypeStruct((NW, L, N_GRAPHS), jnp.float32),
        scratch_types=[
            pltpu.VMEM((B,), jnp.int32),
            pltpu.VMEM((B,), jnp.int32),
            pltpu.VMEM((B,), jnp.int32),
            pltpu.VMEM((B,), jnp.int32),
            pltpu.VMEM((B,), jnp.int32),
            pltpu.VMEM((B,), jnp.int32),
            pltpu.VMEM((B, 8), jnp.float32),
            pltpu.VMEM((B, 8), jnp.float32),
            pltpu.VMEM((B, 8), jnp.float32),
            pltpu.VMEM((B, 8), jnp.float32),
            pltpu.VMEM((B, 8), jnp.float32),
            pltpu.VMEM((L, N_GRAPHS), jnp.float32),
            pltpu.SemaphoreType.DMA,
        ],
        compiler_params=cp,
    )


def kernel(pos, mapping, mapping_batch, atom_types, thetas, ks):
    f32 = jnp.float32
    th = jnp.pad(thetas.reshape(R_TAB, N_DEGS).T.astype(f32),
                 ((0, 0), (0, R_PAD - R_TAB)))
    kk = jnp.pad(ks.reshape(R_TAB, N_DEGS).T.astype(f32),
                 ((0, 0), (0, R_PAD - R_TAB)))
    kc, ksn, ksum = _prep(th, kk)
    ptab = jnp.concatenate(
        [kc[:, :R_TAB].T, ksn[:, :R_TAB].T, ksum[:1, :R_TAB].T,
         jnp.zeros((R_TAB, 1), f32)], axis=1)

    nodes = jnp.concatenate(
        [pos.astype(f32), atom_types.astype(f32)[:, None],
         jnp.zeros((N_NODES, 4), f32)], axis=1)

    mapping = mapping.astype(jnp.int32)
    m0, m1, m2, m3 = mapping[0], mapping[1], mapping[2], mapping[3]
    mb = mapping_batch.astype(jnp.int32)

    partials = _make_sc_kernel()(m0, m1, m2, m3, mb, nodes, ptab)
    y = _final(partials.reshape(NW * L, N_GRAPHS))
    return y[0]


# speedup vs baseline: 216.8553x; 1.3730x over previous
import dataclasses
import functools

import jax
import jax.numpy as jnp
from jax import lax
from jax.experimental import pallas as pl
from jax.experimental.pallas import tpu as pltpu
from jax.experimental.pallas import tpu_sc as plsc

N_NODES = 100000
N_DIH = 3200000
N_TYPES = 26
N_GRAPHS = 64
N_DEGS = 3
R_TAB = N_TYPES ** 4

NC, NS, L = 2, 16, 16
NW = NC * NS
W_PER_TILE = N_DIH // NW
B = 400
N_CHUNKS = W_PER_TILE // B
NV = B // L

PREP_CB = 8192
R_PAD = 458752


def _prep_body(th_ref, k_ref, kc_ref, ks_ref, ksum_ref):
    th = th_ref[...]
    k = k_ref[...]
    kc_ref[...] = k * jnp.cos(th)
    ks_ref[...] = k * jnp.sin(th)
    ksum_ref[...] = jnp.sum(k, axis=0, keepdims=True)


_prep = pl.pallas_call(
    _prep_body,
    grid=(R_PAD // PREP_CB,),
    in_specs=[
        pl.BlockSpec((N_DEGS, PREP_CB), lambda i: (0, i)),
        pl.BlockSpec((N_DEGS, PREP_CB), lambda i: (0, i)),
    ],
    out_specs=[
        pl.BlockSpec((N_DEGS, PREP_CB), lambda i: (0, i)),
        pl.BlockSpec((N_DEGS, PREP_CB), lambda i: (0, i)),
        pl.BlockSpec((1, PREP_CB), lambda i: (0, i)),
    ],
    out_shape=[
        jax.ShapeDtypeStruct((N_DEGS, R_PAD), jnp.float32),
        jax.ShapeDtypeStruct((N_DEGS, R_PAD), jnp.float32),
        jax.ShapeDtypeStruct((1, R_PAD), jnp.float32),
    ],
)


def _final_body(p_ref, o_ref):
    o_ref[...] = jnp.sum(p_ref[...], axis=0, keepdims=True)


_final = pl.pallas_call(
    _final_body,
    out_shape=jax.ShapeDtypeStruct((1, N_GRAPHS), jnp.float32),
)


def _rsqrt(x):
    i = plsc.bitcast(x, jnp.int32)
    i = jnp.int32(0x5F3759DF) - (i >> 1)
    y = plsc.bitcast(i, jnp.float32)
    hx = x * jnp.float32(0.5)
    for _ in range(3):
        y = y * (jnp.float32(1.5) - hx * y * y)
    return y


def _splat_i32(val):
    return jnp.full((L,), val, jnp.int32)


def _sc_body(m0, m1, m2, m3, mb, nodes, ptab, out,
             i0a, i1a, i2a, i3a, gba, fxa, nb0a, nb1a, nb2a, nb3a, pba,
             i0b, i1b, i2b, i3b, gbb, fxb, nb0b, nb1b, nb2b, nb3b, pbb,
             acc, sem_lin, sem_nod, sem_par):
    cid = lax.axis_index("c")
    sid = lax.axis_index("s")
    wid = sid * NC + cid
    base0 = wid * W_PER_TILE

    ring_a = dict(idx=(i0a, i1a, i2a, i3a), gb=gba, fx=fxa,
                  nb=(nb0a, nb1a, nb2a, nb3a), pb=pba)
    ring_b = dict(idx=(i0b, i1b, i2b, i3b), gb=gbb, fx=fxb,
                  nb=(nb0b, nb1b, nb2b, nb3b), pb=pbb)

    for r in range(L):
        for cb in range(N_GRAPHS // L):
            acc[r, pl.ds(cb * L, L)] = jnp.zeros((L,), jnp.float32)

    lane = lax.iota(jnp.int32, L)
    cols = [_splat_i32(v) for v in range(7)]

    def fire_linear(ci, ring):
        base = base0 + ci * B
        for src, dst in zip((m0, m1, m2, m3), ring["idx"]):
            pltpu.async_copy(src.at[pl.ds(base, B)], dst, sem_lin)
        pltpu.async_copy(mb.at[pl.ds(base, B)], ring["gb"], sem_lin)

    def wait_linear(ring):
        for src, dst in zip((m0, m1, m2, m3), ring["idx"]):
            pltpu.make_async_copy(src.at[pl.ds(0, B)], dst, sem_lin).wait()
        pltpu.make_async_copy(mb.at[pl.ds(0, B)], ring["gb"], sem_lin).wait()

    def fire_nodes(ring):
        for ix, dst in zip(ring["idx"], ring["nb"]):
            pltpu.async_copy(nodes.at[ix], dst, sem_nod)

    def wait_nodes(ring):
        for dst in ring["nb"]:
            pltpu.make_async_copy(nodes.at[pl.ds(0, B)], dst, sem_nod).wait()

    def pass1(ring):
        nb0, nb1, nb2, nb3 = ring["nb"]
        fidx = ring["fx"]

        @pl.loop(0, NV)
        def _p1(v):
            rows = lane + v * L
            t0 = plsc.load_gather(nb0, [rows, cols[3]])
            t1 = plsc.load_gather(nb1, [rows, cols[3]])
            t2 = plsc.load_gather(nb2, [rows, cols[3]])
            t3 = plsc.load_gather(nb3, [rows, cols[3]])
            f = ((t0 * jnp.float32(N_TYPES) + t1) * jnp.float32(N_TYPES)
                 + t2) * jnp.float32(N_TYPES) + t3
            f = jnp.minimum(jnp.maximum(f, jnp.float32(0.0)),
                            jnp.float32(R_TAB - 1))
            fidx[pl.ds(v * L, L)] = f.astype(jnp.int32)

    def pass2(ring):
        nb0, nb1, nb2, nb3 = ring["nb"]
        pbuf = ring["pb"]
        gbuf = ring["gb"]

        @pl.loop(0, NV)
        def _p2(v):
            rows = lane + v * L
            p0x = plsc.load_gather(nb0, [rows, cols[0]])
            p0y = plsc.load_gather(nb0, [rows, cols[1]])
            p0z = plsc.load_gather(nb0, [rows, cols[2]])
            p1x = plsc.load_gather(nb1, [rows, cols[0]])
            p1y = plsc.load_gather(nb1, [rows, cols[1]])
            p1z = plsc.load_gather(nb1, [rows, cols[2]])
            p2x = plsc.load_gather(nb2, [rows, cols[0]])
            p2y = plsc.load_gather(nb2, [rows, cols[1]])
            p2z = plsc.load_gather(nb2, [rows, cols[2]])
            p3x = plsc.load_gather(nb3, [rows, cols[0]])
            p3y = plsc.load_gather(nb3, [rows, cols[1]])
            p3z = plsc.load_gather(nb3, [rows, cols[2]])

            b1x, b1y, b1z = p1x - p0x, p1y - p0y, p1z - p0z
            b2x, b2y, b2z = p2x - p1x, p2y - p1y, p2z - p1z
            b3x, b3y, b3z = p3x - p2x, p3y - p2y, p3z - p2z

            n1x = b1y * b2z - b1z * b2y
            n1y = b1z * b2x - b1x * b2z
            n1z = b1x * b2y - b1y * b2x
            n2x = b2y * b3z - b2z * b3y
            n2y = b2z * b3x - b2x * b3z
            n2z = b2x * b3y - b2y * b3x
            mx = n1y * b2z - n1z * b2y
            my = n1z * b2x - n1x * b2z
            mz = n1x * b2y - n1y * b2x

            x = n1x * n2x + n1y * n2y + n1z * n2z
            yp = mx * n2x + my * n2y + mz * n2z
            s2 = jnp.maximum(b2x * b2x + b2y * b2y + b2z * b2z,
                             jnp.float32(1e-30))
            y = yp * _rsqrt(s2)
            r2 = x * x + y * y
            w = _rsqrt(jnp.maximum(r2, jnp.float32(1e-30)))
            dgn = r2 < jnp.float32(1e-30)
            c = jnp.where(dgn, jnp.float32(1.0), x * w)
            s = jnp.where(dgn, jnp.float32(0.0), y * w)

            c2 = jnp.float32(2.0) * c * c - jnp.float32(1.0)
            s2t = jnp.float32(2.0) * s * c
            c3 = jnp.float32(2.0) * c * c2 - c
            s3 = jnp.float32(2.0) * c * s2t - s

            kc1 = plsc.load_gather(pbuf, [rows, cols[0]])
            kc2 = plsc.load_gather(pbuf, [rows, cols[1]])
            kc3 = plsc.load_gather(pbuf, [rows, cols[2]])
            ks1 = plsc.load_gather(pbuf, [rows, cols[3]])
            ks2 = plsc.load_gather(pbuf, [rows, cols[4]])
            ks3 = plsc.load_gather(pbuf, [rows, cols[5]])
            ksm = plsc.load_gather(pbuf, [rows, cols[6]])

            V = ksm - (kc1 * c + kc2 * c2 + kc3 * c3
                       + ks1 * s + ks2 * s2t + ks3 * s3)

            g = gbuf[pl.ds(v * L, L)]
            plsc.addupdate_scatter(acc, [lane, g], V)

    def step(ci, cur, nxt, fire_next_nodes, fire_next2_linear):
        wait_nodes(cur)
        pass1(cur)
        pltpu.async_copy(ptab.at[cur["fx"]], cur["pb"], sem_par)
        if fire_next_nodes:
            wait_linear(nxt)
            fire_nodes(nxt)
        pltpu.make_async_copy(ptab.at[pl.ds(0, B)], cur["pb"], sem_par).wait()
        pass2(cur)
        if fire_next2_linear:
            fire_linear(ci + 2, cur)

    fire_linear(0, ring_a)
    wait_linear(ring_a)
    fire_nodes(ring_a)
    fire_linear(1, ring_b)

    @pl.loop(0, (N_CHUNKS - 2) // 2)
    def _pair(j):
        step(2 * j, ring_a, ring_b, True, True)
        step(2 * j + 1, ring_b, ring_a, True, True)

    step(N_CHUNKS - 2, ring_a, ring_b, True, False)
    step(N_CHUNKS - 1, ring_b, ring_a, False, False)

    pltpu.sync_copy(acc, out.at[wid])


@functools.cache
def _make_sc_kernel():
    mesh = plsc.VectorSubcoreMesh(core_axis_name="c", subcore_axis_name="s")
    cp = pltpu.CompilerParams(use_tc_tiling_on_sc=False,
                              needs_layout_passes=False)
    ring = [
        pltpu.VMEM((B,), jnp.int32),
        pltpu.VMEM((B,), jnp.int32),
        pltpu.VMEM((B,), jnp.int32),
        pltpu.VMEM((B,), jnp.int32),
        pltpu.VMEM((B,), jnp.int32),
        pltpu.VMEM((B,), jnp.int32),
        pltpu.VMEM((B, 8), jnp.float32),
        pltpu.VMEM((B, 8), jnp.float32),
        pltpu.VMEM((B, 8), jnp.float32),
        pltpu.VMEM((B, 8), jnp.float32),
        pltpu.VMEM((B, 8), jnp.float32),
    ]
    return pl.kernel(
        _sc_body,
        mesh=mesh,
        out_type=jax.ShapeDtypeStruct((NW, L, N_GRAPHS), jnp.float32),
        scratch_types=ring + ring + [
            pltpu.VMEM((L, N_GRAPHS), jnp.float32),
            pltpu.SemaphoreType.DMA,
            pltpu.SemaphoreType.DMA,
            pltpu.SemaphoreType.DMA,
        ],
        compiler_params=cp,
    )


def kernel(pos, mapping, mapping_batch, atom_types, thetas, ks):
    f32 = jnp.float32
    th = jnp.pad(thetas.reshape(R_TAB, N_DEGS).T.astype(f32),
                 ((0, 0), (0, R_PAD - R_TAB)))
    kk = jnp.pad(ks.reshape(R_TAB, N_DEGS).T.astype(f32),
                 ((0, 0), (0, R_PAD - R_TAB)))
    kc, ksn, ksum = _prep(th, kk)
    ptab = jnp.concatenate(
        [kc[:, :R_TAB].T, ksn[:, :R_TAB].T, ksum[:1, :R_TAB].T,
         jnp.zeros((R_TAB, 1), f32)], axis=1)

    nodes = jnp.concatenate(
        [pos.astype(f32), atom_types.astype(f32)[:, None],
         jnp.zeros((N_NODES, 4), f32)], axis=1)

    mapping = mapping.astype(jnp.int32)
    m0, m1, m2, m3 = mapping[0], mapping[1], mapping[2], mapping[3]
    mb = mapping_batch.astype(jnp.int32)

    partials = _make_sc_kernel()(m0, m1, m2, m3, mb, nodes, ptab)
    y = _final(partials.reshape(NW * L, N_GRAPHS))
    return y[0]


# speedup vs baseline: 230.2316x; 1.0617x over previous
import dataclasses
import functools

import jax
import jax.numpy as jnp
from jax import lax
from jax.experimental import pallas as pl
from jax.experimental.pallas import tpu as pltpu
from jax.experimental.pallas import tpu_sc as plsc

N_NODES = 100000
N_DIH = 3200000
N_TYPES = 26
N_GRAPHS = 64
N_DEGS = 3
R_TAB = N_TYPES ** 4

NC, NS, L = 2, 16, 16
NW = NC * NS
W_PER_TILE = N_DIH // NW
B = 800
N_CHUNKS = W_PER_TILE // B
NV = B // L

PREP_CB = 8192
R_PAD = 458752


def _prep_body(th_ref, k_ref, kc_ref, ks_ref, ksum_ref):
    th = th_ref[...]
    k = k_ref[...]
    kc_ref[...] = k * jnp.cos(th)
    ks_ref[...] = k * jnp.sin(th)
    ksum_ref[...] = jnp.sum(k, axis=0, keepdims=True)


_prep = pl.pallas_call(
    _prep_body,
    grid=(R_PAD // PREP_CB,),
    in_specs=[
        pl.BlockSpec((N_DEGS, PREP_CB), lambda i: (0, i)),
        pl.BlockSpec((N_DEGS, PREP_CB), lambda i: (0, i)),
    ],
    out_specs=[
        pl.BlockSpec((N_DEGS, PREP_CB), lambda i: (0, i)),
        pl.BlockSpec((N_DEGS, PREP_CB), lambda i: (0, i)),
        pl.BlockSpec((1, PREP_CB), lambda i: (0, i)),
    ],
    out_shape=[
        jax.ShapeDtypeStruct((N_DEGS, R_PAD), jnp.float32),
        jax.ShapeDtypeStruct((N_DEGS, R_PAD), jnp.float32),
        jax.ShapeDtypeStruct((1, R_PAD), jnp.float32),
    ],
)


def _final_body(p_ref, o_ref):
    o_ref[...] = jnp.sum(p_ref[...], axis=0, keepdims=True)


_final = pl.pallas_call(
    _final_body,
    out_shape=jax.ShapeDtypeStruct((1, N_GRAPHS), jnp.float32),
)


def _rsqrt(x):
    i = plsc.bitcast(x, jnp.int32)
    i = jnp.int32(0x5F3759DF) - (i >> 1)
    y = plsc.bitcast(i, jnp.float32)
    hx = x * jnp.float32(0.5)
    for _ in range(3):
        y = y * (jnp.float32(1.5) - hx * y * y)
    return y


def _splat_i32(val):
    return jnp.full((L,), val, jnp.int32)


def _sc_body(m0, m1, m2, m3, mb, nodes, ptab, out,
             i0a, i1a, i2a, i3a, gba, fxa, nb0a, nb1a, nb2a, nb3a, pba,
             i0b, i1b, i2b, i3b, gbb, fxb, nb0b, nb1b, nb2b, nb3b, pbb,
             acc, spn, sem_lin, sem_nod, sem_par):
    cid = lax.axis_index("c")
    sid = lax.axis_index("s")
    wid = sid * NC + cid
    base0 = wid * W_PER_TILE

    ring_a = dict(idx=(i0a, i1a, i2a, i3a), gb=gba, fx=fxa,
                  nb=(nb0a, nb1a, nb2a, nb3a), pb=pba)
    ring_b = dict(idx=(i0b, i1b, i2b, i3b), gb=gbb, fx=fxb,
                  nb=(nb0b, nb1b, nb2b, nb3b), pb=pbb)

    @pl.when(sid == 0)
    def _stage():
        pltpu.sync_copy(nodes, spn)

    plsc.subcore_barrier()

    for r in range(L):
        for cb in range(N_GRAPHS // L):
            acc[r, pl.ds(cb * L, L)] = jnp.zeros((L,), jnp.float32)

    lane = lax.iota(jnp.int32, L)
    cols = [_splat_i32(v) for v in range(7)]

    def fire_linear(ci, ring):
        base = base0 + ci * B
        for src, dst in zip((m0, m1, m2, m3), ring["idx"]):
            pltpu.async_copy(src.at[pl.ds(base, B)], dst, sem_lin)
        pltpu.async_copy(mb.at[pl.ds(base, B)], ring["gb"], sem_lin)

    def wait_linear(ring):
        for src, dst in zip((m0, m1, m2, m3), ring["idx"]):
            pltpu.make_async_copy(src.at[pl.ds(0, B)], dst, sem_lin).wait()
        pltpu.make_async_copy(mb.at[pl.ds(0, B)], ring["gb"], sem_lin).wait()

    def fire_nodes(ring):
        for ix, dst in zip(ring["idx"], ring["nb"]):
            pltpu.async_copy(spn.at[ix], dst, sem_nod)

    def wait_nodes(ring):
        for dst in ring["nb"]:
            pltpu.make_async_copy(spn.at[pl.ds(0, B)], dst, sem_nod).wait()

    def pass1(ring):
        nb0, nb1, nb2, nb3 = ring["nb"]
        fidx = ring["fx"]

        @pl.loop(0, NV)
        def _p1(v):
            rows = lane + v * L
            t0 = plsc.load_gather(nb0, [rows, cols[3]])
            t1 = plsc.load_gather(nb1, [rows, cols[3]])
            t2 = plsc.load_gather(nb2, [rows, cols[3]])
            t3 = plsc.load_gather(nb3, [rows, cols[3]])
            f = ((t0 * jnp.float32(N_TYPES) + t1) * jnp.float32(N_TYPES)
                 + t2) * jnp.float32(N_TYPES) + t3
            f = jnp.minimum(jnp.maximum(f, jnp.float32(0.0)),
                            jnp.float32(R_TAB - 1))
            fidx[pl.ds(v * L, L)] = f.astype(jnp.int32)

    def pass2(ring):
        nb0, nb1, nb2, nb3 = ring["nb"]
        pbuf = ring["pb"]
        gbuf = ring["gb"]

        @pl.loop(0, NV)
        def _p2(v):
            rows = lane + v * L
            p0x = plsc.load_gather(nb0, [rows, cols[0]])
            p0y = plsc.load_gather(nb0, [rows, cols[1]])
            p0z = plsc.load_gather(nb0, [rows, cols[2]])
            p1x = plsc.load_gather(nb1, [rows, cols[0]])
            p1y = plsc.load_gather(nb1, [rows, cols[1]])
            p1z = plsc.load_gather(nb1, [rows, cols[2]])
            p2x = plsc.load_gather(nb2, [rows, cols[0]])
            p2y = plsc.load_gather(nb2, [rows, cols[1]])
            p2z = plsc.load_gather(nb2, [rows, cols[2]])
            p3x = plsc.load_gather(nb3, [rows, cols[0]])
            p3y = plsc.load_gather(nb3, [rows, cols[1]])
            p3z = plsc.load_gather(nb3, [rows, cols[2]])

            b1x, b1y, b1z = p1x - p0x, p1y - p0y, p1z - p0z
            b2x, b2y, b2z = p2x - p1x, p2y - p1y, p2z - p1z
            b3x, b3y, b3z = p3x - p2x, p3y - p2y, p3z - p2z

            n1x = b1y * b2z - b1z * b2y
            n1y = b1z * b2x - b1x * b2z
            n1z = b1x * b2y - b1y * b2x
            n2x = b2y * b3z - b2z * b3y
            n2y = b2z * b3x - b2x * b3z
            n2z = b2x * b3y - b2y * b3x
            mx = n1y * b2z - n1z * b2y
            my = n1z * b2x - n1x * b2z
            mz = n1x * b2y - n1y * b2x

            x = n1x * n2x + n1y * n2y + n1z * n2z
            yp = mx * n2x + my * n2y + mz * n2z
            s2 = jnp.maximum(b2x * b2x + b2y * b2y + b2z * b2z,
                             jnp.float32(1e-30))
            y = yp * _rsqrt(s2)
            r2 = x * x + y * y
            w = _rsqrt(jnp.maximum(r2, jnp.float32(1e-30)))
            dgn = r2 < jnp.float32(1e-30)
            c = jnp.where(dgn, jnp.float32(1.0), x * w)
            s = jnp.where(dgn, jnp.float32(0.0), y * w)

            c2 = jnp.float32(2.0) * c * c - jnp.float32(1.0)
            s2t = jnp.float32(2.0) * s * c
            c3 = jnp.float32(2.0) * c * c2 - c
            s3 = jnp.float32(2.0) * c * s2t - s

            kc1 = plsc.load_gather(pbuf, [rows, cols[0]])
            kc2 = plsc.load_gather(pbuf, [rows, cols[1]])
            kc3 = plsc.load_gather(pbuf, [rows, cols[2]])
            ks1 = plsc.load_gather(pbuf, [rows, cols[3]])
            ks2 = plsc.load_gather(pbuf, [rows, cols[4]])
            ks3 = plsc.load_gather(pbuf, [rows, cols[5]])
            ksm = plsc.load_gather(pbuf, [rows, cols[6]])

            V = ksm - (kc1 * c + kc2 * c2 + kc3 * c3
                       + ks1 * s + ks2 * s2t + ks3 * s3)

            g = gbuf[pl.ds(v * L, L)]
            plsc.addupdate_scatter(acc, [lane, g], V)

    def step(ci, cur, nxt, fire_next_nodes, fire_next2_linear):
        wait_nodes(cur)
        pass1(cur)
        pltpu.async_copy(ptab.at[cur["fx"]], cur["pb"], sem_par)
        if fire_next_nodes:
            wait_linear(nxt)
            fire_nodes(nxt)
        pltpu.make_async_copy(ptab.at[pl.ds(0, B)], cur["pb"], sem_par).wait()
        pass2(cur)
        if fire_next2_linear:
            fire_linear(ci + 2, cur)

    fire_linear(0, ring_a)
    wait_linear(ring_a)
    fire_nodes(ring_a)
    fire_linear(1, ring_b)

    @pl.loop(0, (N_CHUNKS - 3) // 2)
    def _pair(j):
        step(2 * j, ring_a, ring_b, True, True)
        step(2 * j + 1, ring_b, ring_a, True, True)

    step(N_CHUNKS - 3, ring_a, ring_b, True, True)
    step(N_CHUNKS - 2, ring_b, ring_a, True, False)
    step(N_CHUNKS - 1, ring_a, ring_b, False, False)

    pltpu.sync_copy(acc, out.at[wid])


@functools.cache
def _make_sc_kernel():
    mesh = plsc.VectorSubcoreMesh(core_axis_name="c", subcore_axis_name="s")
    cp = pltpu.CompilerParams(use_tc_tiling_on_sc=False,
                              needs_layout_passes=False)
    ring = [
        pltpu.VMEM((B,), jnp.int32),
        pltpu.VMEM((B,), jnp.int32),
        pltpu.VMEM((B,), jnp.int32),
        pltpu.VMEM((B,), jnp.int32),
        pltpu.VMEM((B,), jnp.int32),
        pltpu.VMEM((B,), jnp.int32),
        pltpu.VMEM((B, 8), jnp.float32),
        pltpu.VMEM((B, 8), jnp.float32),
        pltpu.VMEM((B, 8), jnp.float32),
        pltpu.VMEM((B, 8), jnp.float32),
        pltpu.VMEM((B, 8), jnp.float32),
    ]
    return pl.kernel(
        _sc_body,
        mesh=mesh,
        out_type=jax.ShapeDtypeStruct((NW, L, N_GRAPHS), jnp.float32),
        scratch_types=ring + ring + [
            pltpu.VMEM((L, N_GRAPHS), jnp.float32),
            pltpu.VMEM_SHARED((N_NODES, 8), jnp.float32),
            pltpu.SemaphoreType.DMA,
            pltpu.SemaphoreType.DMA,
            pltpu.SemaphoreType.DMA,
        ],
        compiler_params=cp,
    )


def kernel(pos, mapping, mapping_batch, atom_types, thetas, ks):
    f32 = jnp.float32
    th = jnp.pad(thetas.reshape(R_TAB, N_DEGS).T.astype(f32),
                 ((0, 0), (0, R_PAD - R_TAB)))
    kk = jnp.pad(ks.reshape(R_TAB, N_DEGS).T.astype(f32),
                 ((0, 0), (0, R_PAD - R_TAB)))
    kc, ksn, ksum = _prep(th, kk)
    ptab = jnp.concatenate(
        [kc[:, :R_TAB].T, ksn[:, :R_TAB].T, ksum[:1, :R_TAB].T,
         jnp.zeros((R_TAB, 1), f32)], axis=1)

    nodes = jnp.concatenate(
        [pos.astype(f32), atom_types.astype(f32)[:, None],
         jnp.zeros((N_NODES, 4), f32)], axis=1)

    mapping = mapping.astype(jnp.int32)
    m0, m1, m2, m3 = mapping[0], mapping[1], mapping[2], mapping[3]
    mb = mapping_batch.astype(jnp.int32)

    partials = _make_sc_kernel()(m0, m1, m2, m3, mb, nodes, ptab)
    y = _final(partials.reshape(NW * L, N_GRAPHS))
    return y[0]


# speedup vs baseline: 232.1207x; 1.0082x over previous
import dataclasses
import functools

import jax
import jax.numpy as jnp
from jax import lax
from jax.experimental import pallas as pl
from jax.experimental.pallas import tpu as pltpu
from jax.experimental.pallas import tpu_sc as plsc

N_NODES = 100000
N_DIH = 3200000
N_TYPES = 26
N_GRAPHS = 64
N_DEGS = 3
R_TAB = N_TYPES ** 4

NC, NS, L = 2, 16, 16
NW = NC * NS
W_PER_TILE = N_DIH // NW
B = 800
N_CHUNKS = W_PER_TILE // B
NV = B // L

PREP_CB = 8192
R_PAD = 458752


def _prep_body(th_ref, k_ref, kc_ref, ks_ref, ksum_ref):
    th = th_ref[...]
    k = k_ref[...]
    kc_ref[...] = k * jnp.cos(th)
    ks_ref[...] = k * jnp.sin(th)
    ksum_ref[...] = jnp.sum(k, axis=0, keepdims=True)


_prep = pl.pallas_call(
    _prep_body,
    grid=(R_PAD // PREP_CB,),
    in_specs=[
        pl.BlockSpec((N_DEGS, PREP_CB), lambda i: (0, i)),
        pl.BlockSpec((N_DEGS, PREP_CB), lambda i: (0, i)),
    ],
    out_specs=[
        pl.BlockSpec((N_DEGS, PREP_CB), lambda i: (0, i)),
        pl.BlockSpec((N_DEGS, PREP_CB), lambda i: (0, i)),
        pl.BlockSpec((1, PREP_CB), lambda i: (0, i)),
    ],
    out_shape=[
        jax.ShapeDtypeStruct((N_DEGS, R_PAD), jnp.float32),
        jax.ShapeDtypeStruct((N_DEGS, R_PAD), jnp.float32),
        jax.ShapeDtypeStruct((1, R_PAD), jnp.float32),
    ],
)


def _final_body(p_ref, o_ref):
    o_ref[...] = jnp.sum(p_ref[...], axis=0, keepdims=True)


_final = pl.pallas_call(
    _final_body,
    out_shape=jax.ShapeDtypeStruct((1, N_GRAPHS), jnp.float32),
)


def _rsqrt(x):
    i = plsc.bitcast(x, jnp.int32)
    i = jnp.int32(0x5F3759DF) - (i >> 1)
    y = plsc.bitcast(i, jnp.float32)
    hx = x * jnp.float32(0.5)
    for _ in range(3):
        y = y * (jnp.float32(1.5) - hx * y * y)
    return y


def _splat_i32(val):
    return jnp.full((L,), val, jnp.int32)


def _sc_body(m0, m1, m2, m3, mb, nodes, ptab, out,
             ia, gba, fxa, nba,
             ib, gbb, fxb, nbb,
             pbuf, acc, spn, sem_lin, sem_nod, sem_par):
    cid = lax.axis_index("c")
    sid = lax.axis_index("s")
    wid = sid * NC + cid
    base0 = wid * W_PER_TILE

    ring_a = dict(idx=ia, gb=gba, fx=fxa, nb=nba)
    ring_b = dict(idx=ib, gb=gbb, fx=fxb, nb=nbb)

    @pl.when(sid == 0)
    def _stage():
        pltpu.sync_copy(nodes, spn)

    plsc.subcore_barrier()

    for r in range(L):
        for cb in range(N_GRAPHS // L):
            acc[r, pl.ds(cb * L, L)] = jnp.zeros((L,), jnp.float32)

    lane = lax.iota(jnp.int32, L)
    cols = [_splat_i32(v) for v in range(7)]

    def fire_linear(ci, ring):
        base = base0 + ci * B
        for d, src in enumerate((m0, m1, m2, m3)):
            pltpu.async_copy(src.at[pl.ds(base, B)],
                             ring["idx"].at[pl.ds(d * B, B)], sem_lin)
        pltpu.async_copy(mb.at[pl.ds(base, B)], ring["gb"], sem_lin)

    def wait_linear(ring):
        for d, src in enumerate((m0, m1, m2, m3)):
            pltpu.make_async_copy(src.at[pl.ds(0, B)],
                                  ring["idx"].at[pl.ds(d * B, B)],
                                  sem_lin).wait()
        pltpu.make_async_copy(mb.at[pl.ds(0, B)], ring["gb"], sem_lin).wait()

    def fire_nodes(ring):
        pltpu.async_copy(spn.at[ring["idx"]], ring["nb"], sem_nod)

    def wait_nodes(ring):
        pltpu.make_async_copy(spn.at[pl.ds(0, 4 * B)], ring["nb"],
                              sem_nod).wait()

    def pass1(ring):
        nb = ring["nb"]
        fidx = ring["fx"]

        @pl.loop(0, NV)
        def _p1(v):
            rows = lane + v * L
            t0 = plsc.load_gather(nb, [rows, cols[3]])
            t1 = plsc.load_gather(nb, [rows + B, cols[3]])
            t2 = plsc.load_gather(nb, [rows + 2 * B, cols[3]])
            t3 = plsc.load_gather(nb, [rows + 3 * B, cols[3]])
            f = ((t0 * jnp.float32(N_TYPES) + t1) * jnp.float32(N_TYPES)
                 + t2) * jnp.float32(N_TYPES) + t3
            f = jnp.minimum(jnp.maximum(f, jnp.float32(0.0)),
                            jnp.float32(R_TAB - 1))
            fidx[pl.ds(v * L, L)] = f.astype(jnp.int32)

    def pass2(ring, v0, nv):
        nb = ring["nb"]
        gbuf = ring["gb"]

        @pl.loop(v0, v0 + nv)
        def _p2(v):
            rows = lane + v * L
            r1 = rows + B
            r2 = rows + 2 * B
            r3 = rows + 3 * B
            p0x = plsc.load_gather(nb, [rows, cols[0]])
            p0y = plsc.load_gather(nb, [rows, cols[1]])
            p0z = plsc.load_gather(nb, [rows, cols[2]])
            p1x = plsc.load_gather(nb, [r1, cols[0]])
            p1y = plsc.load_gather(nb, [r1, cols[1]])
            p1z = plsc.load_gather(nb, [r1, cols[2]])
            p2x = plsc.load_gather(nb, [r2, cols[0]])
            p2y = plsc.load_gather(nb, [r2, cols[1]])
            p2z = plsc.load_gather(nb, [r2, cols[2]])
            p3x = plsc.load_gather(nb, [r3, cols[0]])
            p3y = plsc.load_gather(nb, [r3, cols[1]])
            p3z = plsc.load_gather(nb, [r3, cols[2]])

            b1x, b1y, b1z = p1x - p0x, p1y - p0y, p1z - p0z
            b2x, b2y, b2z = p2x - p1x, p2y - p1y, p2z - p1z
            b3x, b3y, b3z = p3x - p2x, p3y - p2y, p3z - p2z

            n1x = b1y * b2z - b1z * b2y
            n1y = b1z * b2x - b1x * b2z
            n1z = b1x * b2y - b1y * b2x
            n2x = b2y * b3z - b2z * b3y
            n2y = b2z * b3x - b2x * b3z
            n2z = b2x * b3y - b2y * b3x
            mx = n1y * b2z - n1z * b2y
            my = n1z * b2x - n1x * b2z
            mz = n1x * b2y - n1y * b2x

            x = n1x * n2x + n1y * n2y + n1z * n2z
            yp = mx * n2x + my * n2y + mz * n2z
            s2 = jnp.maximum(b2x * b2x + b2y * b2y + b2z * b2z,
                             jnp.float32(1e-30))
            y = yp * _rsqrt(s2)
            r2 = x * x + y * y
            w = _rsqrt(jnp.maximum(r2, jnp.float32(1e-30)))
            dgn = r2 < jnp.float32(1e-30)
            c = jnp.where(dgn, jnp.float32(1.0), x * w)
            s = jnp.where(dgn, jnp.float32(0.0), y * w)

            c2 = jnp.float32(2.0) * c * c - jnp.float32(1.0)
            s2t = jnp.float32(2.0) * s * c
            c3 = jnp.float32(2.0) * c * c2 - c
            s3 = jnp.float32(2.0) * c * s2t - s

            kc1 = plsc.load_gather(pbuf, [rows, cols[0]])
            kc2 = plsc.load_gather(pbuf, [rows, cols[1]])
            kc3 = plsc.load_gather(pbuf, [rows, cols[2]])
            ks1 = plsc.load_gather(pbuf, [rows, cols[3]])
            ks2 = plsc.load_gather(pbuf, [rows, cols[4]])
            ks3 = plsc.load_gather(pbuf, [rows, cols[5]])
            ksm = plsc.load_gather(pbuf, [rows, cols[6]])

            V = ksm - (kc1 * c + kc2 * c2 + kc3 * c3
                       + ks1 * s + ks2 * s2t + ks3 * s3)

            g = gbuf[pl.ds(v * L, L)]
            plsc.addupdate_scatter(acc, [lane, g], V)

    H = B // 2
    HV = NV // 2

    def step(ci, cur, nxt, fire_next_nodes, fire_next2_linear):
        wait_nodes(cur)
        pass1(cur)
        pltpu.async_copy(ptab.at[cur["fx"].at[pl.ds(0, H)]],
                         pbuf.at[pl.ds(0, H)], sem_par)
        pltpu.async_copy(ptab.at[cur["fx"].at[pl.ds(H, H)]],
                         pbuf.at[pl.ds(H, H)], sem_par)
        if fire_next_nodes:
            wait_linear(nxt)
            fire_nodes(nxt)
        pltpu.make_async_copy(ptab.at[pl.ds(0, H)],
                              pbuf.at[pl.ds(0, H)], sem_par).wait()
        pass2(cur, 0, HV)
        if fire_next2_linear:
            fire_linear(ci + 2, cur)
        pltpu.make_async_copy(ptab.at[pl.ds(0, H)],
                              pbuf.at[pl.ds(H, H)], sem_par).wait()
        pass2(cur, HV, HV)

    fire_linear(0, ring_a)
    wait_linear(ring_a)
    fire_nodes(ring_a)
    fire_linear(1, ring_b)

    @pl.loop(0, (N_CHUNKS - 3) // 2)
    def _pair(j):
        step(2 * j, ring_a, ring_b, True, True)
        step(2 * j + 1, ring_b, ring_a, True, True)

    step(N_CHUNKS - 3, ring_a, ring_b, True, True)
    step(N_CHUNKS - 2, ring_b, ring_a, True, False)
    step(N_CHUNKS - 1, ring_a, ring_b, False, False)

    pltpu.sync_copy(acc, out.at[wid])


@functools.cache
def _make_sc_kernel():
    mesh = plsc.VectorSubcoreMesh(core_axis_name="c", subcore_axis_name="s")
    cp = pltpu.CompilerParams(use_tc_tiling_on_sc=False,
                              needs_layout_passes=False)
    ring = [
        pltpu.VMEM((4 * B,), jnp.int32),
        pltpu.VMEM((B,), jnp.int32),
        pltpu.VMEM((B,), jnp.int32),
        pltpu.VMEM((4 * B, 8), jnp.float32),
    ]
    return pl.kernel(
        _sc_body,
        mesh=mesh,
        out_type=jax.ShapeDtypeStruct((NW, L, N_GRAPHS), jnp.float32),
        scratch_types=ring + ring + [
            pltpu.VMEM((B, 8), jnp.float32),
            pltpu.VMEM((L, N_GRAPHS), jnp.float32),
            pltpu.VMEM_SHARED((N_NODES, 8), jnp.float32),
            pltpu.SemaphoreType.DMA,
            pltpu.SemaphoreType.DMA,
            pltpu.SemaphoreType.DMA,
        ],
        compiler_params=cp,
    )


def kernel(pos, mapping, mapping_batch, atom_types, thetas, ks):
    f32 = jnp.float32
    th = jnp.pad(thetas.reshape(R_TAB, N_DEGS).T.astype(f32),
                 ((0, 0), (0, R_PAD - R_TAB)))
    kk = jnp.pad(ks.reshape(R_TAB, N_DEGS).T.astype(f32),
                 ((0, 0), (0, R_PAD - R_TAB)))
    kc, ksn, ksum = _prep(th, kk)
    ptab = jnp.concatenate(
        [kc[:, :R_TAB].T, ksn[:, :R_TAB].T, ksum[:1, :R_TAB].T,
         jnp.zeros((R_TAB, 1), f32)], axis=1)

    nodes = jnp.concatenate(
        [pos.astype(f32), atom_types.astype(f32)[:, None],
         jnp.zeros((N_NODES, 4), f32)], axis=1)

    mapping = mapping.astype(jnp.int32)
    m0, m1, m2, m3 = mapping[0], mapping[1], mapping[2], mapping[3]
    mb = mapping_batch.astype(jnp.int32)

    partials = _make_sc_kernel()(m0, m1, m2, m3, mb, nodes, ptab)
    y = _final(partials.reshape(NW * L, N_GRAPHS))
    return y[0]
